# baseline (speedup 1.0000x reference)
import functools

import jax
import jax.numpy as jnp
from jax import lax
from jax.experimental import pallas as pl
from jax.experimental.pallas import tpu as pltpu
from jax.experimental.pallas import tpu_sc as plsc

_NC = 2
_NS = 16
_NW = _NC * _NS

_GW = 128


def _proj_body(x_ref, w_ref, b_ref, pd_ref, ps_ref):
    rh = pd_ref.shape[1]
    out = jnp.dot(x_ref[...], w_ref[...], preferred_element_type=jnp.float32)
    out = out + b_ref[...]
    pd_ref[...] = out[:, :rh]
    ps_ref[...] = out[:, rh:]


def _proj(x, wcat, bcat, bn):
    n = x.shape[0]
    nd = x.shape[1]
    rh = wcat.shape[1] // 2
    grid = n // bn
    return pl.pallas_call(
        _proj_body,
        grid=(grid,),
        in_specs=[
            pl.BlockSpec((bn, nd), lambda i: (i, 0)),
            pl.BlockSpec((nd, 2 * rh), lambda i: (0, 0)),
            pl.BlockSpec((1, 2 * rh), lambda i: (0, 0)),
        ],
        out_specs=[
            pl.BlockSpec((bn, rh), lambda i: (i, 0)),
            pl.BlockSpec((bn, rh), lambda i: (i, 0)),
        ],
        out_shape=[
            jax.ShapeDtypeStruct((n, rh), jnp.float32),
            jax.ShapeDtypeStruct((n, rh), jnp.float32),
        ],
    )(x, wcat, bcat)


def _gather(pd, ps, dst, src, chunk):
    e_num = dst.shape[0]
    rh = pd.shape[1]
    per_w = e_num // _NW
    n_ch = per_w // chunk
    assert n_ch % 2 == 0 and n_ch >= 4 and chunk % 8 == 0
    mesh = plsc.VectorSubcoreMesh(
        core_axis_name="c", subcore_axis_name="s",
        num_cores=_NC, num_subcores=_NS)

    @functools.partial(
        pl.kernel,
        out_type=jax.ShapeDtypeStruct((e_num, _GW), jnp.float32),
        mesh=mesh,
        compiler_params=pltpu.CompilerParams(use_tc_tiling_on_sc=False),
        scratch_types=[
            pltpu.VMEM((2, chunk), jnp.int32),
            pltpu.VMEM((2, chunk), jnp.int32),
            pltpu.VMEM((2, chunk, rh), jnp.float32),
            pltpu.VMEM((2, chunk, rh), jnp.float32),
            pltpu.SemaphoreType.DMA,
            pltpu.SemaphoreType.DMA,
            pltpu.SemaphoreType.DMA,
            pltpu.SemaphoreType.DMA,
            pltpu.SemaphoreType.DMA,
            pltpu.SemaphoreType.DMA,
        ],
    )
    def k(pd_hbm, ps_hbm, dst_hbm, src_hbm, g_hbm,
          idxd_v, idxs_v, rowd_v, rows_v, semi0, semi1, semg0, semg1,
          semo0, semo1):
        wid = lax.axis_index("s") * _NC + lax.axis_index("c")
        w0 = wid * per_w
        semi = (semi0, semi1)
        semg = (semg0, semg1)
        semo = (semo0, semo1)

        def issue_idx(ch, b):
            base = w0 + ch * chunk
            pltpu.async_copy(dst_hbm.at[pl.ds(base, chunk)], idxd_v.at[b],
                             semi[b])
            pltpu.async_copy(src_hbm.at[pl.ds(base, chunk)], idxs_v.at[b],
                             semi[b])

        def wait_idx(b):
            pltpu.make_async_copy(dst_hbm.at[pl.ds(0, chunk)], idxd_v.at[b],
                                  semi[b]).wait()
            pltpu.make_async_copy(src_hbm.at[pl.ds(0, chunk)], idxs_v.at[b],
                                  semi[b]).wait()

        def wait_out(b):
            pltpu.make_async_copy(
                rowd_v.at[b], g_hbm.at[pl.ds(0, chunk), pl.ds(0, rh)],
                semo[b]).wait()
            pltpu.make_async_copy(
                rows_v.at[b], g_hbm.at[pl.ds(0, chunk), pl.ds(rh, rh)],
                semo[b]).wait()

        issue_idx(0, 0)
        issue_idx(1, 1)

        def group_body(g, carry):
            for b in range(2):
                ch = g * 2 + b
                base = w0 + ch * chunk
                wait_idx(b)

                @pl.when(ch >= 2)
                def _():
                    wait_out(b)

                cpd = pltpu.async_copy(
                    pd_hbm.at[idxd_v.at[b]], rowd_v.at[b], semg[b])
                cps = pltpu.async_copy(
                    ps_hbm.at[idxs_v.at[b]], rows_v.at[b], semg[b])
                cpd.wait()
                cps.wait()

                @pl.when(ch + 2 < n_ch)
                def _():
                    issue_idx(ch + 2, b)
                pltpu.async_copy(
                    rowd_v.at[b],
                    g_hbm.at[pl.ds(base, chunk), pl.ds(0, rh)], semo[b])
                pltpu.async_copy(
                    rows_v.at[b],
                    g_hbm.at[pl.ds(base, chunk), pl.ds(rh, rh)], semo[b])
            return carry

        lax.fori_loop(0, n_ch // 2, group_body, 0)
        wait_out(0)
        wait_out(1)

    return k(pd, ps, dst, src)


def _edge_body_packed(g_ref, e_ref, webig_ref, wr2big_ref, br2cat_ref, m_ref):
    rh = webig_ref.shape[1] // 8
    gsums = [g_ref[j][:, :rh] + g_ref[j][:, rh:2 * rh] for j in range(8)]
    h = jnp.concatenate(gsums, axis=1)
    h = h + jnp.dot(e_ref[...], webig_ref[...],
                    preferred_element_type=jnp.float32)
    h = jax.nn.relu(h)
    m_ref[...] = (
        jnp.dot(h, wr2big_ref[...], preferred_element_type=jnp.float32)
        + br2cat_ref[...])


def _edge_mlp_packed(g3, e_pack, webig, wr2big, br2cat, be8):
    e8 = e_pack.shape[0]
    ed8 = e_pack.shape[1]
    rh8 = webig.shape[1]
    grid = e8 // be8
    return pl.pallas_call(
        _edge_body_packed,
        grid=(grid,),
        in_specs=[
            pl.BlockSpec((8, be8, _GW), lambda i: (0, i, 0)),
            pl.BlockSpec((be8, ed8), lambda i: (i, 0)),
            pl.BlockSpec((ed8, rh8), lambda i: (0, 0)),
            pl.BlockSpec((rh8, ed8), lambda i: (0, 0)),
            pl.BlockSpec((1, ed8), lambda i: (0, 0)),
        ],
        out_specs=pl.BlockSpec((be8, ed8), lambda i: (i, 0)),
        out_shape=jax.ShapeDtypeStruct((e8, ed8), jnp.float32),
    )(g3, e_pack, webig, wr2big, br2cat)


def _split_ranges(e8, frac_num=3, frac_den=5):
    r1 = (e8 * frac_num // frac_den) // 8 * 8
    return r1, e8 - r1


def _pack_e_body(*refs):
    out_ref = refs[-1]
    out_ref[...] = jnp.concatenate([r[...] for r in refs[:-1]], axis=1)


def _pack_e(edge_attr, be8):
    e_num, ed = edge_attr.shape
    e8 = e_num // 8
    nb = e8 // be8
    return pl.pallas_call(
        _pack_e_body,
        grid=(nb,),
        in_specs=[
            pl.BlockSpec((be8, ed), lambda i, j=j: (j * nb + i, 0))
            for j in range(8)
        ],
        out_specs=pl.BlockSpec((be8, 8 * ed), lambda i: (i, 0)),
        out_shape=jax.ShapeDtypeStruct((e8, 8 * ed), jnp.float32),
    )(*([edge_attr] * 8))


def _scatter(m, dst, zeros_rows, n_nodes, chunk):
    e_num, ed = m.shape
    per_w = e_num // _NW
    n_ch = per_w // chunk
    zc = zeros_rows.shape[0]
    nz = n_nodes // zc
    mesh = plsc.VectorSubcoreMesh(
        core_axis_name="c", subcore_axis_name="s",
        num_cores=_NC, num_subcores=_NS)

    @functools.partial(
        pl.kernel,
        out_type=jax.ShapeDtypeStruct((_NC, n_nodes, ed), jnp.float32),
        mesh=mesh,
        compiler_params=pltpu.CompilerParams(use_tc_tiling_on_sc=False),
        scratch_types=[
            pltpu.VMEM((2, chunk), jnp.int32),
            pltpu.VMEM((2, chunk, ed), jnp.float32),
            pltpu.VMEM_SHARED((n_nodes, ed), jnp.float32),
            pltpu.SemaphoreType.DMA,
            pltpu.SemaphoreType.DMA,
        ],
    )
    def k(m_hbm, dst_hbm, z_hbm, agg_hbm, idx_v, rows_v, acc_s, semi0, semi1):
        cid = lax.axis_index("c")
        sid = lax.axis_index("s")
        wid = sid * _NC + cid
        w0 = wid * per_w
        semi = (semi0, semi1)

        @pl.when(sid < nz)
        def _():
            pltpu.sync_copy(z_hbm, acc_s.at[pl.ds(sid * zc, zc)])

        def issue_in(ch, b):
            base = w0 + ch * chunk
            pltpu.async_copy(dst_hbm.at[pl.ds(base, chunk)], idx_v.at[b],
                             semi[b])
            pltpu.async_copy(m_hbm.at[pl.ds(base, chunk)], rows_v.at[b],
                             semi[b])

        def wait_in(b):
            pltpu.make_async_copy(dst_hbm.at[pl.ds(0, chunk)], idx_v.at[b],
                                  semi[b]).wait()
            pltpu.make_async_copy(m_hbm.at[pl.ds(0, chunk)], rows_v.at[b],
                                  semi[b]).wait()

        issue_in(0, 0)
        issue_in(1, 1)
        plsc.subcore_barrier()

        def group_body(g, carry):
            for b in range(2):
                ch = g * 2 + b
                wait_in(b)
                pltpu.sync_copy(rows_v.at[b], acc_s.at[idx_v.at[b]], add=True)

                @pl.when(ch + 2 < n_ch)
                def _():
                    issue_in(ch + 2, b)
            return carry

        lax.fori_loop(0, n_ch // 2, group_body, 0)

        plsc.subcore_barrier()

        @pl.when(sid < nz)
        def _():
            pltpu.sync_copy(acc_s.at[pl.ds(sid * zc, zc)],
                            agg_hbm.at[cid, pl.ds(sid * zc, zc)])

    return k(m, dst, zeros_rows)


def _node_body(x_ref, a0_ref, a1_ref, a2_ref, a3_ref, wox_ref, woa_ref,
               bo1_ref, wo2_ref, bo2_ref, out_ref):
    x = x_ref[...]
    agg = (a0_ref[...] + a1_ref[...]) + (a2_ref[...] + a3_ref[...])
    nh = jnp.dot(x, wox_ref[...], preferred_element_type=jnp.float32)
    nh = nh + jnp.dot(agg, woa_ref[...], preferred_element_type=jnp.float32)
    nh = jax.nn.relu(nh + bo1_ref[...])
    dx = jnp.dot(nh, wo2_ref[...], preferred_element_type=jnp.float32)
    dx = dx + bo2_ref[...]
    out_ref[...] = 0.5 * x + 0.5 * jax.nn.relu(dx)


def _node_update(x, aggs, wox, woa, bo1, wo2, bo2, bn):
    n, nd = x.shape
    ed = aggs[0].shape[1]
    oh = wox.shape[1]
    grid = n // bn
    return pl.pallas_call(
        _node_body,
        grid=(grid,),
        in_specs=[
            pl.BlockSpec((bn, nd), lambda i: (i, 0)),
            pl.BlockSpec((bn, ed), lambda i: (i, 0)),
            pl.BlockSpec((bn, ed), lambda i: (i, 0)),
            pl.BlockSpec((bn, ed), lambda i: (i, 0)),
            pl.BlockSpec((bn, ed), lambda i: (i, 0)),
            pl.BlockSpec((nd, oh), lambda i: (0, 0)),
            pl.BlockSpec((ed, oh), lambda i: (0, 0)),
            pl.BlockSpec((1, oh), lambda i: (0, 0)),
            pl.BlockSpec((oh, nd), lambda i: (0, 0)),
            pl.BlockSpec((1, nd), lambda i: (0, 0)),
        ],
        out_specs=pl.BlockSpec((bn, nd), lambda i: (i, 0)),
        out_shape=jax.ShapeDtypeStruct((n, nd), jnp.float32),
    )(x, *aggs, wox, woa, bo1, wo2, bo2)


def _node_proj_body(x_ref, a0_ref, a1_ref, a2_ref, a3_ref, wox_ref, woa_ref,
                    bo1_ref, wo2_ref, bo2_ref, wcat_ref, bcat_ref,
                    out_ref, pd_ref, ps_ref):
    rh = pd_ref.shape[1]
    x = x_ref[...]
    agg = (a0_ref[...] + a1_ref[...]) + (a2_ref[...] + a3_ref[...])
    nh = jnp.dot(x, wox_ref[...], preferred_element_type=jnp.float32)
    nh = nh + jnp.dot(agg, woa_ref[...], preferred_element_type=jnp.float32)
    nh = jax.nn.relu(nh + bo1_ref[...])
    dx = jnp.dot(nh, wo2_ref[...], preferred_element_type=jnp.float32)
    dx = dx + bo2_ref[...]
    xn = 0.5 * x + 0.5 * jax.nn.relu(dx)
    out_ref[...] = xn
    pp = jnp.dot(xn, wcat_ref[...], preferred_element_type=jnp.float32)
    pp = pp + bcat_ref[...]
    pd_ref[...] = pp[:, :rh]
    ps_ref[...] = pp[:, rh:]


def _node_update_proj(x, aggs, wox, woa, bo1, wo2, bo2, wcat, bcat, bn):
    n, nd = x.shape
    ed = aggs[0].shape[1]
    oh = wox.shape[1]
    rh = wcat.shape[1] // 2
    grid = n // bn
    return pl.pallas_call(
        _node_proj_body,
        grid=(grid,),
        in_specs=[
            pl.BlockSpec((bn, nd), lambda i: (i, 0)),
            pl.BlockSpec((bn, ed), lambda i: (i, 0)),
            pl.BlockSpec((bn, ed), lambda i: (i, 0)),
            pl.BlockSpec((bn, ed), lambda i: (i, 0)),
            pl.BlockSpec((bn, ed), lambda i: (i, 0)),
            pl.BlockSpec((nd, oh), lambda i: (0, 0)),
            pl.BlockSpec((ed, oh), lambda i: (0, 0)),
            pl.BlockSpec((1, oh), lambda i: (0, 0)),
            pl.BlockSpec((oh, nd), lambda i: (0, 0)),
            pl.BlockSpec((1, nd), lambda i: (0, 0)),
            pl.BlockSpec((nd, 2 * rh), lambda i: (0, 0)),
            pl.BlockSpec((1, 2 * rh), lambda i: (0, 0)),
        ],
        out_specs=[
            pl.BlockSpec((bn, nd), lambda i: (i, 0)),
            pl.BlockSpec((bn, rh), lambda i: (i, 0)),
            pl.BlockSpec((bn, rh), lambda i: (i, 0)),
        ],
        out_shape=[
            jax.ShapeDtypeStruct((n, nd), jnp.float32),
            jax.ShapeDtypeStruct((n, rh), jnp.float32),
            jax.ShapeDtypeStruct((n, rh), jnp.float32),
        ],
    )(x, *aggs, wox, woa, bo1, wo2, bo2, wcat, bcat)


def kernel(x, edge_index, edge_attr, Wr1, br1, Wr2, br2, Wo1, bo1, Wo2, bo2):
    n, nd = x.shape
    e_num, ed = edge_attr.shape
    num_layers = Wr1.shape[0]
    oh = Wo1.shape[2]

    src = edge_index[0]
    dst = edge_index[1]

    bn = 1000
    be = 8000
    g_chunk = 200
    s_chunk = 1000
    zc = 1000

    zeros_rows = jnp.zeros((zc, ed), jnp.float32)

    def wcat_bcat(l):
        wcat = jnp.concatenate([Wr1[l, :nd], Wr1[l, nd:2 * nd]], axis=1)
        bcat = jnp.concatenate(
            [br1[l], jnp.zeros_like(br1[l])]).reshape(1, -1)
        return wcat, bcat

    e8 = e_num // 8
    rh = Wr1.shape[2]
    r1, r2 = _split_ranges(e8)
    e_a, e_b = 8 * r1, 8 * r2
    dst8 = jnp.reshape(dst, (8, e8))
    src8 = jnp.reshape(src, (8, e8))
    dst_a = jnp.reshape(dst8[:, :r1], (e_a,))
    src_a = jnp.reshape(src8[:, :r1], (e_a,))
    dst_b = jnp.reshape(dst8[:, r1:], (e_b,))
    src_b = jnp.reshape(src8[:, r1:], (e_b,))
    dst_a_s = jnp.reshape(jnp.transpose(dst8[:, :r1]), (e_a,))
    dst_b_s = jnp.reshape(jnp.transpose(dst8[:, r1:]), (e_b,))

    ep = _pack_e(edge_attr, be // 8)
    ea_pack, eb_pack = ep[:r1], ep[r1:]
    wcat, bcat = wcat_bcat(0)
    pd, ps = _proj(x, wcat, bcat, bn)
    for l in range(num_layers):
        we = Wr1[l, 2 * nd:]
        eye8 = jnp.eye(8, dtype=jnp.float32)
        webig = jnp.kron(eye8, we)
        wr2big = jnp.kron(eye8, Wr2[l])
        br2cat = jnp.tile(br2[l], 8).reshape(1, 8 * ed)

        g_a = _gather(pd, ps, dst_a, src_a, g_chunk)
        g_b = _gather(pd, ps, dst_b, src_b, g_chunk)
        m_a = _edge_mlp_packed(jnp.reshape(g_a, (8, r1, _GW)), ea_pack,
                               webig, wr2big, br2cat, be // 8)
        agg_a = _scatter(jnp.reshape(m_a, (e_a, ed)), dst_a_s,
                         zeros_rows, n, s_chunk)
        m_b = _edge_mlp_packed(jnp.reshape(g_b, (8, r2, _GW)), eb_pack,
                               webig, wr2big, br2cat, be // 8)
        agg_b = _scatter(jnp.reshape(m_b, (e_b, ed)), dst_b_s,
                         zeros_rows, n, s_chunk)
        aggs = (agg_a[0], agg_a[1], agg_b[0], agg_b[1])
        if l + 1 < num_layers:
            wcat, bcat = wcat_bcat(l + 1)
            x, pd, ps = _node_update_proj(
                x, aggs,
                Wo1[l, :nd], Wo1[l, nd:], bo1[l].reshape(1, oh),
                Wo2[l], bo2[l].reshape(1, nd), wcat, bcat, bn)
        else:
            x = _node_update(x, aggs,
                             Wo1[l, :nd], Wo1[l, nd:], bo1[l].reshape(1, oh),
                             Wo2[l], bo2[l].reshape(1, nd), bn)
        ea_pack, eb_pack = m_a, m_b
    t_a = jnp.transpose(jnp.reshape(m_a, (r1, 8, ed)), (1, 0, 2))
    t_b = jnp.transpose(jnp.reshape(m_b, (r2, 8, ed)), (1, 0, 2))
    e_out = jnp.reshape(jnp.concatenate([t_a, t_b], axis=1), (e_num, ed))
    return x, e_out


# speedup vs baseline: 1.0421x; 1.0421x over previous
import functools

import jax
import jax.numpy as jnp
from jax import lax
from jax.experimental import pallas as pl
from jax.experimental.pallas import tpu as pltpu
from jax.experimental.pallas import tpu_sc as plsc

_NC = 2
_NS = 16
_NW = _NC * _NS

_GW = 128


def _proj_body(x_ref, w_ref, b_ref, pd_ref, ps_ref):
    rh = pd_ref.shape[1]
    out = jnp.dot(x_ref[...], w_ref[...], preferred_element_type=jnp.float32)
    out = out + b_ref[...]
    pd_ref[...] = out[:, :rh]
    ps_ref[...] = out[:, rh:]


def _proj(x, wcat, bcat, bn):
    n = x.shape[0]
    nd = x.shape[1]
    rh = wcat.shape[1] // 2
    grid = n // bn
    return pl.pallas_call(
        _proj_body,
        grid=(grid,),
        in_specs=[
            pl.BlockSpec((bn, nd), lambda i: (i, 0)),
            pl.BlockSpec((nd, 2 * rh), lambda i: (0, 0)),
            pl.BlockSpec((1, 2 * rh), lambda i: (0, 0)),
        ],
        out_specs=[
            pl.BlockSpec((bn, rh), lambda i: (i, 0)),
            pl.BlockSpec((bn, rh), lambda i: (i, 0)),
        ],
        out_shape=[
            jax.ShapeDtypeStruct((n, rh), jnp.float32),
            jax.ShapeDtypeStruct((n, rh), jnp.float32),
        ],
    )(x, wcat, bcat)


def _gather(pd, ps, dst, src, chunk):
    e_num = dst.shape[0]
    rh = pd.shape[1]
    per_w = e_num // _NW
    n_ch = per_w // chunk
    assert n_ch % 2 == 0 and n_ch >= 4 and chunk % 8 == 0
    mesh = plsc.VectorSubcoreMesh(
        core_axis_name="c", subcore_axis_name="s",
        num_cores=_NC, num_subcores=_NS)

    @functools.partial(
        pl.kernel,
        out_type=jax.ShapeDtypeStruct((e_num, _GW), jnp.float32),
        mesh=mesh,
        compiler_params=pltpu.CompilerParams(use_tc_tiling_on_sc=False),
        scratch_types=[
            pltpu.VMEM((2, chunk), jnp.int32),
            pltpu.VMEM((2, chunk), jnp.int32),
            pltpu.VMEM((2, chunk, rh), jnp.float32),
            pltpu.VMEM((2, chunk, rh), jnp.float32),
            pltpu.SemaphoreType.DMA,
            pltpu.SemaphoreType.DMA,
            pltpu.SemaphoreType.DMA,
            pltpu.SemaphoreType.DMA,
            pltpu.SemaphoreType.DMA,
            pltpu.SemaphoreType.DMA,
        ],
    )
    def k(pd_hbm, ps_hbm, dst_hbm, src_hbm, g_hbm,
          idxd_v, idxs_v, rowd_v, rows_v, semi0, semi1, semg0, semg1,
          semo0, semo1):
        wid = lax.axis_index("s") * _NC + lax.axis_index("c")
        w0 = wid * per_w
        semi = (semi0, semi1)
        semg = (semg0, semg1)
        semo = (semo0, semo1)

        def issue_idx(ch, b):
            base = w0 + ch * chunk
            pltpu.async_copy(dst_hbm.at[pl.ds(base, chunk)], idxd_v.at[b],
                             semi[b])
            pltpu.async_copy(src_hbm.at[pl.ds(base, chunk)], idxs_v.at[b],
                             semi[b])

        def wait_idx(b):
            pltpu.make_async_copy(dst_hbm.at[pl.ds(0, chunk)], idxd_v.at[b],
                                  semi[b]).wait()
            pltpu.make_async_copy(src_hbm.at[pl.ds(0, chunk)], idxs_v.at[b],
                                  semi[b]).wait()

        def wait_out(b):
            pltpu.make_async_copy(
                rowd_v.at[b], g_hbm.at[pl.ds(0, chunk), pl.ds(0, rh)],
                semo[b]).wait()
            pltpu.make_async_copy(
                rows_v.at[b], g_hbm.at[pl.ds(0, chunk), pl.ds(rh, rh)],
                semo[b]).wait()

        issue_idx(0, 0)
        issue_idx(1, 1)

        def group_body(g, carry):
            for b in range(2):
                ch = g * 2 + b
                base = w0 + ch * chunk
                wait_idx(b)

                @pl.when(ch >= 2)
                def _():
                    wait_out(b)

                cpd = pltpu.async_copy(
                    pd_hbm.at[idxd_v.at[b]], rowd_v.at[b], semg[b])
                cps = pltpu.async_copy(
                    ps_hbm.at[idxs_v.at[b]], rows_v.at[b], semg[b])
                cpd.wait()
                cps.wait()

                @pl.when(ch + 2 < n_ch)
                def _():
                    issue_idx(ch + 2, b)
                pltpu.async_copy(
                    rowd_v.at[b],
                    g_hbm.at[pl.ds(base, chunk), pl.ds(0, rh)], semo[b])
                pltpu.async_copy(
                    rows_v.at[b],
                    g_hbm.at[pl.ds(base, chunk), pl.ds(rh, rh)], semo[b])
            return carry

        lax.fori_loop(0, n_ch // 2, group_body, 0)
        wait_out(0)
        wait_out(1)

    return k(pd, ps, dst, src)


def _edge_body_packed(g_ref, e_ref, webig_ref, wr2big_ref, br2cat_ref, m_ref):
    rh = webig_ref.shape[1] // 8
    gsums = [g_ref[j][:, :rh] + g_ref[j][:, rh:2 * rh] for j in range(8)]
    h = jnp.concatenate(gsums, axis=1)
    h = h + jnp.dot(e_ref[...], webig_ref[...],
                    preferred_element_type=jnp.float32)
    h = jax.nn.relu(h)
    m_ref[...] = (
        jnp.dot(h, wr2big_ref[...], preferred_element_type=jnp.float32)
        + br2cat_ref[...])


def _edge_mlp_packed(g3, e_pack, webig, wr2big, br2cat, be8):
    e8 = e_pack.shape[0]
    ed8 = e_pack.shape[1]
    rh8 = webig.shape[1]
    grid = e8 // be8
    return pl.pallas_call(
        _edge_body_packed,
        grid=(grid,),
        in_specs=[
            pl.BlockSpec((8, be8, _GW), lambda i: (0, i, 0)),
            pl.BlockSpec((be8, ed8), lambda i: (i, 0)),
            pl.BlockSpec((ed8, rh8), lambda i: (0, 0)),
            pl.BlockSpec((rh8, ed8), lambda i: (0, 0)),
            pl.BlockSpec((1, ed8), lambda i: (0, 0)),
        ],
        out_specs=pl.BlockSpec((be8, ed8), lambda i: (i, 0)),
        out_shape=jax.ShapeDtypeStruct((e8, ed8), jnp.float32),
    )(g3, e_pack, webig, wr2big, br2cat)


def _pack_e_body(*refs):
    out_ref = refs[-1]
    out_ref[...] = jnp.concatenate([r[...] for r in refs[:-1]], axis=1)


def _pack_e(edge_attr, be8):
    e_num, ed = edge_attr.shape
    e8 = e_num // 8
    nb = e8 // be8
    return pl.pallas_call(
        _pack_e_body,
        grid=(nb,),
        in_specs=[
            pl.BlockSpec((be8, ed), lambda i, j=j: (j * nb + i, 0))
            for j in range(8)
        ],
        out_specs=pl.BlockSpec((be8, 8 * ed), lambda i: (i, 0)),
        out_shape=jax.ShapeDtypeStruct((e8, 8 * ed), jnp.float32),
    )(*([edge_attr] * 8))


def _scatter(m, dst, zeros_rows, n_nodes, chunk):
    e_num, ed = m.shape
    per_w = e_num // _NW
    n_ch = per_w // chunk
    zc = zeros_rows.shape[0]
    nz = n_nodes // zc
    mesh = plsc.VectorSubcoreMesh(
        core_axis_name="c", subcore_axis_name="s",
        num_cores=_NC, num_subcores=_NS)

    @functools.partial(
        pl.kernel,
        out_type=jax.ShapeDtypeStruct((_NC, n_nodes, ed), jnp.float32),
        mesh=mesh,
        compiler_params=pltpu.CompilerParams(use_tc_tiling_on_sc=False),
        scratch_types=[
            pltpu.VMEM((2, chunk), jnp.int32),
            pltpu.VMEM((2, chunk, ed), jnp.float32),
            pltpu.VMEM_SHARED((n_nodes, ed), jnp.float32),
            pltpu.SemaphoreType.DMA,
            pltpu.SemaphoreType.DMA,
        ],
    )
    def k(m_hbm, dst_hbm, z_hbm, agg_hbm, idx_v, rows_v, acc_s, semi0, semi1):
        cid = lax.axis_index("c")
        sid = lax.axis_index("s")
        wid = sid * _NC + cid
        w0 = wid * per_w
        semi = (semi0, semi1)

        @pl.when(sid < nz)
        def _():
            pltpu.sync_copy(z_hbm, acc_s.at[pl.ds(sid * zc, zc)])

        def issue_in(ch, b):
            base = w0 + ch * chunk
            pltpu.async_copy(dst_hbm.at[pl.ds(base, chunk)], idx_v.at[b],
                             semi[b])
            pltpu.async_copy(m_hbm.at[pl.ds(base, chunk)], rows_v.at[b],
                             semi[b])

        def wait_in(b):
            pltpu.make_async_copy(dst_hbm.at[pl.ds(0, chunk)], idx_v.at[b],
                                  semi[b]).wait()
            pltpu.make_async_copy(m_hbm.at[pl.ds(0, chunk)], rows_v.at[b],
                                  semi[b]).wait()

        issue_in(0, 0)
        issue_in(1, 1)
        plsc.subcore_barrier()

        def group_body(g, carry):
            for b in range(2):
                ch = g * 2 + b
                wait_in(b)
                pltpu.sync_copy(rows_v.at[b], acc_s.at[idx_v.at[b]], add=True)

                @pl.when(ch + 2 < n_ch)
                def _():
                    issue_in(ch + 2, b)
            return carry

        lax.fori_loop(0, n_ch // 2, group_body, 0)

        plsc.subcore_barrier()

        @pl.when(sid < nz)
        def _():
            pltpu.sync_copy(acc_s.at[pl.ds(sid * zc, zc)],
                            agg_hbm.at[cid, pl.ds(sid * zc, zc)])

    return k(m, dst, zeros_rows)


def _node_body(x_ref, a0_ref, a1_ref, wox_ref, woa_ref, bo1_ref,
               wo2_ref, bo2_ref, out_ref):
    x = x_ref[...]
    agg = a0_ref[...] + a1_ref[...]
    nh = jnp.dot(x, wox_ref[...], preferred_element_type=jnp.float32)
    nh = nh + jnp.dot(agg, woa_ref[...], preferred_element_type=jnp.float32)
    nh = jax.nn.relu(nh + bo1_ref[...])
    dx = jnp.dot(nh, wo2_ref[...], preferred_element_type=jnp.float32)
    dx = dx + bo2_ref[...]
    out_ref[...] = 0.5 * x + 0.5 * jax.nn.relu(dx)


def _node_update(x, a0, a1, wox, woa, bo1, wo2, bo2, bn):
    n, nd = x.shape
    ed = a0.shape[1]
    oh = wox.shape[1]
    grid = n // bn
    return pl.pallas_call(
        _node_body,
        grid=(grid,),
        in_specs=[
            pl.BlockSpec((bn, nd), lambda i: (i, 0)),
            pl.BlockSpec((bn, ed), lambda i: (i, 0)),
            pl.BlockSpec((bn, ed), lambda i: (i, 0)),
            pl.BlockSpec((nd, oh), lambda i: (0, 0)),
            pl.BlockSpec((ed, oh), lambda i: (0, 0)),
            pl.BlockSpec((1, oh), lambda i: (0, 0)),
            pl.BlockSpec((oh, nd), lambda i: (0, 0)),
            pl.BlockSpec((1, nd), lambda i: (0, 0)),
        ],
        out_specs=pl.BlockSpec((bn, nd), lambda i: (i, 0)),
        out_shape=jax.ShapeDtypeStruct((n, nd), jnp.float32),
    )(x, a0, a1, wox, woa, bo1, wo2, bo2)


def _node_proj_body(x_ref, a0_ref, a1_ref, wox_ref, woa_ref, bo1_ref,
                    wo2_ref, bo2_ref, wcat_ref, bcat_ref,
                    out_ref, pd_ref, ps_ref):
    rh = pd_ref.shape[1]
    x = x_ref[...]
    agg = a0_ref[...] + a1_ref[...]
    nh = jnp.dot(x, wox_ref[...], preferred_element_type=jnp.float32)
    nh = nh + jnp.dot(agg, woa_ref[...], preferred_element_type=jnp.float32)
    nh = jax.nn.relu(nh + bo1_ref[...])
    dx = jnp.dot(nh, wo2_ref[...], preferred_element_type=jnp.float32)
    dx = dx + bo2_ref[...]
    xn = 0.5 * x + 0.5 * jax.nn.relu(dx)
    out_ref[...] = xn
    pp = jnp.dot(xn, wcat_ref[...], preferred_element_type=jnp.float32)
    pp = pp + bcat_ref[...]
    pd_ref[...] = pp[:, :rh]
    ps_ref[...] = pp[:, rh:]


def _node_update_proj(x, a0, a1, wox, woa, bo1, wo2, bo2, wcat, bcat, bn):
    n, nd = x.shape
    ed = a0.shape[1]
    oh = wox.shape[1]
    rh = wcat.shape[1] // 2
    grid = n // bn
    return pl.pallas_call(
        _node_proj_body,
        grid=(grid,),
        in_specs=[
            pl.BlockSpec((bn, nd), lambda i: (i, 0)),
            pl.BlockSpec((bn, ed), lambda i: (i, 0)),
            pl.BlockSpec((bn, ed), lambda i: (i, 0)),
            pl.BlockSpec((nd, oh), lambda i: (0, 0)),
            pl.BlockSpec((ed, oh), lambda i: (0, 0)),
            pl.BlockSpec((1, oh), lambda i: (0, 0)),
            pl.BlockSpec((oh, nd), lambda i: (0, 0)),
            pl.BlockSpec((1, nd), lambda i: (0, 0)),
            pl.BlockSpec((nd, 2 * rh), lambda i: (0, 0)),
            pl.BlockSpec((1, 2 * rh), lambda i: (0, 0)),
        ],
        out_specs=[
            pl.BlockSpec((bn, nd), lambda i: (i, 0)),
            pl.BlockSpec((bn, rh), lambda i: (i, 0)),
            pl.BlockSpec((bn, rh), lambda i: (i, 0)),
        ],
        out_shape=[
            jax.ShapeDtypeStruct((n, nd), jnp.float32),
            jax.ShapeDtypeStruct((n, rh), jnp.float32),
            jax.ShapeDtypeStruct((n, rh), jnp.float32),
        ],
    )(x, a0, a1, wox, woa, bo1, wo2, bo2, wcat, bcat)


def kernel(x, edge_index, edge_attr, Wr1, br1, Wr2, br2, Wo1, bo1, Wo2, bo2):
    n, nd = x.shape
    e_num, ed = edge_attr.shape
    num_layers = Wr1.shape[0]
    oh = Wo1.shape[2]

    src = edge_index[0]
    dst = edge_index[1]

    bn = 1000
    be = 16000
    g_chunk = 200
    s_chunk = 1000
    zc = 1000

    zeros_rows = jnp.zeros((zc, ed), jnp.float32)

    def wcat_bcat(l):
        wcat = jnp.concatenate([Wr1[l, :nd], Wr1[l, nd:2 * nd]], axis=1)
        bcat = jnp.concatenate(
            [br1[l], jnp.zeros_like(br1[l])]).reshape(1, -1)
        return wcat, bcat

    e8 = e_num // 8
    rh = Wr1.shape[2]
    dst_s = jnp.reshape(jnp.transpose(jnp.reshape(dst, (8, e8))), (e_num,))

    e_pack = _pack_e(edge_attr, be // 8)
    wcat, bcat = wcat_bcat(0)
    pd, ps = _proj(x, wcat, bcat, bn)
    for l in range(num_layers):
        we = Wr1[l, 2 * nd:]
        eye8 = jnp.eye(8, dtype=jnp.float32)
        webig = jnp.kron(eye8, we)
        wr2big = jnp.kron(eye8, Wr2[l])
        br2cat = jnp.tile(br2[l], 8).reshape(1, 8 * ed)

        g = _gather(pd, ps, dst, src, g_chunk)
        g3 = jnp.reshape(g, (8, e8, _GW))
        m_pack = _edge_mlp_packed(g3, e_pack, webig, wr2big, br2cat,
                                  be // 8)
        m = jnp.reshape(m_pack, (e_num, ed))
        agg2 = _scatter(m, dst_s, zeros_rows, n, s_chunk)
        if l + 1 < num_layers:
            wcat, bcat = wcat_bcat(l + 1)
            x, pd, ps = _node_update_proj(
                x, agg2[0], agg2[1],
                Wo1[l, :nd], Wo1[l, nd:], bo1[l].reshape(1, oh),
                Wo2[l], bo2[l].reshape(1, nd), wcat, bcat, bn)
        else:
            x = _node_update(x, agg2[0], agg2[1],
                             Wo1[l, :nd], Wo1[l, nd:], bo1[l].reshape(1, oh),
                             Wo2[l], bo2[l].reshape(1, nd), bn)
        e_pack = m_pack
        e_out = m
    e_out = jnp.reshape(
        jnp.transpose(jnp.reshape(e_out, (e8, 8, ed)), (1, 0, 2)),
        (e_num, ed))
    return x, e_out
